# baseline (speedup 1.0000x reference)
import functools

import jax
import jax.numpy as jnp
from jax import lax
from jax.experimental import pallas as pl
from jax.experimental.pallas import tpu as pltpu
from jax.experimental.pallas import tpu_sc as plsc

_LS = 0.1
_BLK_W = 2048
_C0 = 51200


def _sc_info():
    try:
        info = plsc.get_sparse_core_info()
        return info.num_cores, info.num_subcores
    except Exception:
        return 2, 16


def _tc_body(label_ref, pred_ref, s_out, m_out, g_out, z_out,
             s_acc, m_acc, g_acc, z_acc):
    j = pl.program_id(0)
    nblk = pl.num_programs(0)
    blk_b, blk_w = pred_ref.shape

    @pl.when(j == 0)
    def _init():
        s_acc[...] = jnp.zeros_like(s_acc)
        m_acc[...] = jnp.full_like(m_acc, -jnp.inf)
        g_acc[...] = jnp.zeros_like(g_acc)
        z_acc[...] = jnp.zeros_like(z_acc)

    x = pred_ref[...]
    lab_local = label_ref[...] - j * blk_w
    match = lax.broadcasted_iota(jnp.int32, (blk_b, blk_w), 1) == lab_local
    g_acc[...] += jnp.sum(jnp.where(match, x, 0.0), axis=1, keepdims=True)
    s_acc[...] += jnp.sum(jnp.exp(x), axis=1, keepdims=True)
    m_acc[...] = jnp.maximum(m_acc[...], jnp.max(x, axis=1, keepdims=True))
    z_acc[...] += jnp.sum(jnp.where(x == 0.0, 1.0, 0.0), axis=1, keepdims=True)

    @pl.when(j == nblk - 1)
    def _emit():
        s_out[...] = s_acc[...]
        m_out[...] = m_acc[...]
        g_out[...] = g_acc[...]
        z_out[...] = z_acc[...]


def _tc_pass(pred, label2d, c0):
    b, _ = pred.shape
    nblk = c0 // _BLK_W
    part = jax.ShapeDtypeStruct((b, 1), jnp.float32)
    return pl.pallas_call(
        _tc_body,
        grid=(nblk,),
        in_specs=[
            pl.BlockSpec((b, 1), lambda j: (0, 0)),
            pl.BlockSpec((b, _BLK_W), lambda j: (0, j)),
        ],
        out_specs=[pl.BlockSpec((b, 1), lambda j: (0, 0))] * 4,
        out_shape=[part] * 4,
        scratch_shapes=[pltpu.VMEM((b, 1), jnp.float32)] * 4,
    )(label2d, pred)


def _sc_pass(pred, label, c0):
    b, c = pred.shape
    k = c - c0
    nv = k // 16
    assert k % 16 == 0 and c0 % 8 == 0
    nc, ns = _sc_info()
    nw = nc * ns
    bw = b // nw
    mesh = plsc.VectorSubcoreMesh(core_axis_name="c", subcore_axis_name="s")
    part = jax.ShapeDtypeStruct((b, 16), jnp.float32)

    @functools.partial(
        pl.kernel,
        mesh=mesh,
        out_type=(part, part, part, part),
        scratch_types=[
            pltpu.VMEM((bw, 16), jnp.int32),
            pltpu.VMEM((k,), jnp.float32),
            pltpu.VMEM((k,), jnp.float32),
            pltpu.VMEM((bw, 16), jnp.float32),
            pltpu.VMEM((bw, 16), jnp.float32),
            pltpu.VMEM((bw, 16), jnp.float32),
            pltpu.VMEM((bw, 16), jnp.float32),
            pltpu.SemaphoreType.DMA,
            pltpu.SemaphoreType.DMA,
        ],
    )
    def sck(pred_hbm, label_hbm, s_hbm, m_hbm, g_hbm, z_hbm,
            lab_v, buf0, buf1, s_o, m_o, g_o, z_o, sem0, sem1):
        wid = lax.axis_index("s") * nc + lax.axis_index("c")
        base = wid * bw
        pltpu.sync_copy(label_hbm.at[pl.ds(base, bw)], lab_v)
        bufs = (buf0, buf1)
        sems = (sem0, sem1)
        cps = [None, None]
        cps[0] = pltpu.async_copy(
            pred_hbm.at[base, pl.ds(c0, k)], buf0, sem0)
        for i in range(bw):
            if i + 1 < bw:
                cps[(i + 1) % 2] = pltpu.async_copy(
                    pred_hbm.at[base + i + 1, pl.ds(c0, k)],
                    bufs[(i + 1) % 2], sems[(i + 1) % 2])
            cps[i % 2].wait()
            buf = bufs[i % 2]
            lab_vec = lab_v[i]

            def step(kk, carry):
                s, m, z, g, col = carry
                v = buf[pl.ds(kk * 16, 16)]
                s = s + jnp.exp(v)
                m = jnp.maximum(m, v)
                z = z + jnp.where(v == 0.0, 1.0, 0.0)
                g = g + jnp.where(col == lab_vec, v, 0.0)
                return s, m, z, g, col + 16

            init = (jnp.zeros((16,), jnp.float32),
                    jnp.full((16,), -jnp.inf, jnp.float32),
                    jnp.zeros((16,), jnp.float32),
                    jnp.zeros((16,), jnp.float32),
                    c0 + lax.iota(jnp.int32, 16))
            s, m, z, g, _ = lax.fori_loop(0, nv, step, init, unroll=8)
            s_o[i] = s
            m_o[i] = m
            z_o[i] = z
            g_o[i] = g
        pltpu.sync_copy(s_o, s_hbm.at[pl.ds(base, bw)])
        pltpu.sync_copy(m_o, m_hbm.at[pl.ds(base, bw)])
        pltpu.sync_copy(g_o, g_hbm.at[pl.ds(base, bw)])
        pltpu.sync_copy(z_o, z_hbm.at[pl.ds(base, bw)])

    lab16 = jnp.broadcast_to(label.astype(jnp.int32)[:, None], (b, 16))
    return sck(pred, lab16)


def _combine_body(s1, m1, g1, z1, s2, m2, g2, z2, out_ref):
    s = s1[...] + jnp.sum(s2[...], axis=1, keepdims=True)
    m = jnp.maximum(m1[...], jnp.max(m2[...], axis=1, keepdims=True))
    g = g1[...] + jnp.sum(g2[...], axis=1, keepdims=True)
    z = z1[...] + jnp.sum(z2[...], axis=1, keepdims=True)
    masked = jnp.exp(m) > 0.5 * s
    at_label = masked & (g == m)
    base = -(1.0 - _LS) * jnp.log(jnp.where(at_label, 1.0 - g, g))
    extra = jnp.where(masked & jnp.logical_not(at_label),
                      -_LS * jnp.log(1.0 - m), 0.0)
    loss = jnp.mean(base + extra)
    z_nonlabel = z - jnp.where(g == 0.0, 1.0, 0.0)
    has_nan = jnp.max(z_nonlabel) > 0.0
    out_ref[...] = jnp.full((1, 1),
                            jnp.where(has_nan, jnp.float32(jnp.nan), loss))


def _combine(tc_parts, sc_parts):
    b = tc_parts[0].shape[0]
    return pl.pallas_call(
        _combine_body,
        out_shape=jax.ShapeDtypeStruct((1, 1), jnp.float32),
    )(*tc_parts, *sc_parts)


@jax.jit
def kernel(pred, label):
    b, c = pred.shape
    label2d = label.reshape(b, 1).astype(jnp.int32)
    tc_parts = _tc_pass(pred, label2d, _C0)
    sc_parts = _sc_pass(pred, label, _C0)
    return _combine(tc_parts, sc_parts).reshape(())


# speedup vs baseline: 1.0020x; 1.0020x over previous
import functools

import jax
import jax.numpy as jnp
from jax import lax
from jax.experimental import pallas as pl
from jax.experimental.pallas import tpu as pltpu
from jax.experimental.pallas import tpu_sc as plsc

_LS = 0.1
_BLK_W = 2048
_C0 = 51200


def _sc_info():
    try:
        info = plsc.get_sparse_core_info()
        return info.num_cores, info.num_subcores
    except Exception:
        return 2, 16


def _tc_body(label_ref, pred_ref, s_out, m_out, g_out, z_out,
             s_acc, m_acc, g_acc, z_acc):
    j = pl.program_id(0)
    nblk = pl.num_programs(0)
    blk_b, blk_w = pred_ref.shape

    @pl.when(j == 0)
    def _init():
        s_acc[...] = jnp.zeros_like(s_acc)
        m_acc[...] = jnp.full_like(m_acc, -jnp.inf)
        g_acc[...] = jnp.zeros_like(g_acc)
        z_acc[...] = jnp.zeros_like(z_acc)

    x = pred_ref[...]
    lab_local = label_ref[...] - j * blk_w
    match = lax.broadcasted_iota(jnp.int32, (blk_b, blk_w), 1) == lab_local
    g_acc[...] += jnp.sum(jnp.where(match, x, 0.0), axis=1, keepdims=True)
    s_acc[...] += jnp.sum(jnp.exp(x), axis=1, keepdims=True)
    m_acc[...] = jnp.maximum(m_acc[...], jnp.max(x, axis=1, keepdims=True))
    z_acc[...] += jnp.sum(jnp.where(x == 0.0, 1.0, 0.0), axis=1, keepdims=True)

    @pl.when(j == nblk - 1)
    def _emit():
        s_out[...] = s_acc[...]
        m_out[...] = m_acc[...]
        g_out[...] = g_acc[...]
        z_out[...] = z_acc[...]


def _tc_pass(pred, label2d, c0):
    b, _ = pred.shape
    nblk = c0 // _BLK_W
    part = jax.ShapeDtypeStruct((b, 1), jnp.float32)
    return pl.pallas_call(
        _tc_body,
        grid=(nblk,),
        in_specs=[
            pl.BlockSpec((b, 1), lambda j: (0, 0)),
            pl.BlockSpec((b, _BLK_W), lambda j: (0, j)),
        ],
        out_specs=[pl.BlockSpec((b, 1), lambda j: (0, 0))] * 4,
        out_shape=[part] * 4,
        scratch_shapes=[pltpu.VMEM((b, 1), jnp.float32)] * 4,
    )(label2d, pred)


def _sc_pass(pred, label, c0):
    b, c = pred.shape
    k = c - c0
    nv = k // 16
    assert k % 16 == 0 and c0 % 8 == 0
    nc, ns = _sc_info()
    nw = nc * ns
    bw = b // nw
    mesh = plsc.VectorSubcoreMesh(core_axis_name="c", subcore_axis_name="s")
    part = jax.ShapeDtypeStruct((b, 16), jnp.float32)

    @functools.partial(
        pl.kernel,
        mesh=mesh,
        out_type=(part, part, part, part),
        scratch_types=[
            pltpu.VMEM((bw, 16), jnp.int32),
            pltpu.VMEM((k,), jnp.float32),
            pltpu.VMEM((k,), jnp.float32),
            pltpu.VMEM((bw, 16), jnp.float32),
            pltpu.VMEM((bw, 16), jnp.float32),
            pltpu.VMEM((bw, 16), jnp.float32),
            pltpu.VMEM((bw, 16), jnp.float32),
            pltpu.SemaphoreType.DMA,
            pltpu.SemaphoreType.DMA,
        ],
    )
    def sck(pred_hbm, label_hbm, s_hbm, m_hbm, g_hbm, z_hbm,
            lab_v, buf0, buf1, s_o, m_o, g_o, z_o, sem0, sem1):
        wid = lax.axis_index("s") * nc + lax.axis_index("c")
        base = wid * bw
        pltpu.sync_copy(label_hbm.at[pl.ds(base, bw)], lab_v)
        bufs = (buf0, buf1)
        sems = (sem0, sem1)
        cps = [None, None]
        cps[0] = pltpu.async_copy(
            pred_hbm.at[base, pl.ds(c0, k)], buf0, sem0)
        for i in range(bw):
            if i + 1 < bw:
                cps[(i + 1) % 2] = pltpu.async_copy(
                    pred_hbm.at[base + i + 1, pl.ds(c0, k)],
                    bufs[(i + 1) % 2], sems[(i + 1) % 2])
            cps[i % 2].wait()
            buf = bufs[i % 2]
            lab_vec = lab_v[i]

            def step(kk, carry):
                s, m, z, g, col = carry
                v = buf[pl.ds(kk * 16, 16)]
                s = s + jnp.exp(v)
                m = jnp.maximum(m, v)
                z = z + jnp.where(v == 0.0, 1.0, 0.0)
                g = g + jnp.where(col == lab_vec, v, 0.0)
                return s, m, z, g, col + 16

            init = (jnp.zeros((16,), jnp.float32),
                    jnp.full((16,), -jnp.inf, jnp.float32),
                    jnp.zeros((16,), jnp.float32),
                    jnp.zeros((16,), jnp.float32),
                    c0 + lax.iota(jnp.int32, 16))
            s, m, z, g, _ = lax.fori_loop(0, nv, step, init, unroll=8)
            s_o[i] = s
            m_o[i] = m
            z_o[i] = z
            g_o[i] = g
        pltpu.sync_copy(s_o, s_hbm.at[pl.ds(base, bw)])
        pltpu.sync_copy(m_o, m_hbm.at[pl.ds(base, bw)])
        pltpu.sync_copy(g_o, g_hbm.at[pl.ds(base, bw)])
        pltpu.sync_copy(z_o, z_hbm.at[pl.ds(base, bw)])

    lab16 = jnp.broadcast_to(label.astype(jnp.int32)[:, None], (b, 16))
    return sck(pred, lab16)


def _combine_body(s1, m1, g1, z1, s2, m2, g2, z2, out_ref):
    s = s1[...] + jnp.sum(s2[...], axis=1, keepdims=True)
    m = jnp.maximum(m1[...], jnp.max(m2[...], axis=1, keepdims=True))
    g = g1[...] + jnp.sum(g2[...], axis=1, keepdims=True)
    z = z1[...] + jnp.sum(z2[...], axis=1, keepdims=True)
    masked = jnp.exp(m) > 0.5 * s
    at_label = masked & (g == m)
    base = -(1.0 - _LS) * jnp.log(jnp.where(at_label, 1.0 - g, g))
    extra = jnp.where(masked & jnp.logical_not(at_label),
                      -_LS * jnp.log(1.0 - m), 0.0)
    loss = jnp.mean(base + extra)
    z_nonlabel = z - jnp.where(g == 0.0, 1.0, 0.0)
    has_nan = jnp.max(z_nonlabel) > 0.0
    out_ref[...] = jnp.full((1, 1),
                            jnp.where(has_nan, jnp.float32(jnp.nan), loss))


def _combine(tc_parts, sc_parts):
    b = tc_parts[0].shape[0]
    return pl.pallas_call(
        _combine_body,
        out_shape=jax.ShapeDtypeStruct((1, 1), jnp.float32),
    )(*tc_parts, *sc_parts)


@jax.jit
def kernel(pred, label):
    b, c = pred.shape
    label2d = label.reshape(b, 1).astype(jnp.int32)
    sc_parts = _sc_pass(pred, label, _C0)
    tc_parts = _tc_pass(pred, label2d, _C0)
    return _combine(tc_parts, sc_parts).reshape(())
